# baseline (speedup 1.0000x reference)
import jax
import jax.numpy as jnp
from jax import lax
from jax.experimental import pallas as pl
from jax.experimental.pallas import tpu as pltpu

_B, _S, _H = 4, 4096, 4096
_E = 64
_RH = 512
_RH2 = 256
_LBW = 0.01
_NTOK = _B * _S
_BT = 1024
_GRID = _NTOK // _BT


def _router_kernel(x_ref, w1_ref, b1_ref, w2_ref, b2_ref, w3_ref, b3_ref,
                   rw_ref, usage_ref, conf_ref, lbl_ref, ent_ref, util_ref,
                   logits_ref):
    step = pl.program_id(0)

    @pl.when(step == 0)
    def _init():
        usage_ref[...] = jnp.zeros_like(usage_ref)
        conf_ref[...] = jnp.zeros_like(conf_ref)

    @pl.when(step > 0)
    def _gate():
        logits = logits_ref[...]
        m = jnp.max(logits, axis=1, keepdims=True)
        ex = jnp.exp(logits - m)
        p = ex / jnp.sum(ex, axis=1, keepdims=True)
        iota = lax.broadcasted_iota(jnp.int32, (_BT, _E), 1)
        m1 = jnp.max(p, axis=1, keepdims=True)
        i1 = jnp.min(jnp.where(p == m1, iota, _E), axis=1, keepdims=True)
        p_rest = jnp.where(iota == i1, -1.0, p)
        m2 = jnp.max(p_rest, axis=1, keepdims=True)
        i2 = jnp.min(jnp.where(p_rest == m2, iota, _E), axis=1, keepdims=True)
        keep = (iota == i1) | (iota == i2)
        denom = m1 + m2 + 1e-8
        rw = jnp.where(keep, p, 0.0) / denom
        rw_ref[...] = rw
        usage_ref[...] += jnp.sum(rw, axis=0, keepdims=True)
        conf_ref[...] += jnp.sum(m1 / denom, axis=0, keepdims=True)

    @pl.when(step < _GRID)
    def _mlp():
        x = x_ref[...]
        h1 = jnp.maximum(
            jnp.dot(x, w1_ref[...], preferred_element_type=jnp.float32)
            + b1_ref[...], 0.0)
        h2 = jnp.maximum(
            jnp.dot(h1, w2_ref[...], preferred_element_type=jnp.float32)
            + b2_ref[...], 0.0)
        logits_ref[...] = (
            jnp.dot(h2, w3_ref[...], preferred_element_type=jnp.float32)
            + b3_ref[...])

    @pl.when(step == _GRID)
    def _finalize():
        usage = usage_ref[...]
        probs = usage * (1.0 / _NTOK)
        util_ref[...] = probs
        d = usage - (_NTOK / _E)
        lbl_ref[...] = jnp.sum(d * d, axis=1, keepdims=True) * (_LBW / _E)
        ent_ref[...] = -jnp.sum(probs * jnp.log(probs + 1e-8), axis=1,
                                keepdims=True)
        conf_ref[...] = conf_ref[...] * (1.0 / _NTOK)


def kernel(hidden_states, W1, b1, W2, b2, W3, b3):
    x = hidden_states.reshape(_NTOK, _H)
    b1r = b1.reshape(1, _RH)
    b2r = b2.reshape(1, _RH2)
    b3r = b3.reshape(1, _E)
    out_shape = (
        jax.ShapeDtypeStruct((_NTOK, _E), jnp.float32),
        jax.ShapeDtypeStruct((1, _E), jnp.float32),
        jax.ShapeDtypeStruct((1, 1), jnp.float32),
        jax.ShapeDtypeStruct((1, 1), jnp.float32),
        jax.ShapeDtypeStruct((1, 1), jnp.float32),
        jax.ShapeDtypeStruct((1, _E), jnp.float32),
    )
    last = _GRID - 1
    in_specs = [
        pl.BlockSpec((_BT, _H), lambda i: (jnp.minimum(i, last), 0)),
        pl.BlockSpec((_H, _RH), lambda i: (0, 0)),
        pl.BlockSpec((1, _RH), lambda i: (0, 0)),
        pl.BlockSpec((_RH, _RH2), lambda i: (0, 0)),
        pl.BlockSpec((1, _RH2), lambda i: (0, 0)),
        pl.BlockSpec((_RH2, _E), lambda i: (0, 0)),
        pl.BlockSpec((1, _E), lambda i: (0, 0)),
    ]
    out_specs = (
        pl.BlockSpec((_BT, _E), lambda i: (jnp.maximum(i - 1, 0), 0)),
        pl.BlockSpec((1, _E), lambda i: (0, 0)),
        pl.BlockSpec((1, 1), lambda i: (0, 0)),
        pl.BlockSpec((1, 1), lambda i: (0, 0)),
        pl.BlockSpec((1, 1), lambda i: (0, 0)),
        pl.BlockSpec((1, _E), lambda i: (0, 0)),
    )
    rw, usage, conf, lbl, ent, util = pl.pallas_call(
        _router_kernel,
        grid=(_GRID + 1,),
        in_specs=in_specs,
        out_specs=out_specs,
        out_shape=out_shape,
        scratch_shapes=[pltpu.VMEM((_BT, _E), jnp.float32)],
        compiler_params=pltpu.CompilerParams(
            dimension_semantics=("arbitrary",)),
    )(x, W1, b1r, W2, b2r, W3, b3r)
    return (rw.reshape(_B, _S, _E), lbl.reshape(()), ent.reshape(()),
            util.reshape(_E), conf.reshape(()), usage.reshape(_E))


# speedup vs baseline: 1.0708x; 1.0708x over previous
import jax
import jax.numpy as jnp
from jax import lax
from jax.experimental import pallas as pl
from jax.experimental.pallas import tpu as pltpu

_B, _S, _H = 4, 4096, 4096
_E = 64
_RH = 512
_RH2 = 256
_LBW = 0.01
_NTOK = _B * _S
_BT = 1024
_GRID = _NTOK // _BT


def _router_kernel(x_ref, w1_ref, b1_ref, w2_ref, b2_ref, w3_ref, b3_ref,
                   rw_ref, usage_ref, conf_ref, lbl_ref, ent_ref, util_ref):
    step = pl.program_id(0)

    @pl.when(step == 0)
    def _init():
        usage_ref[...] = jnp.zeros_like(usage_ref)
        conf_ref[...] = jnp.zeros_like(conf_ref)

    x = x_ref[...]
    h1 = jnp.maximum(
        jnp.dot(x, w1_ref[...], preferred_element_type=jnp.float32)
        + b1_ref[...], 0.0)
    h2 = jnp.maximum(
        jnp.dot(h1, w2_ref[...], preferred_element_type=jnp.float32)
        + b2_ref[...], 0.0)
    logits = (jnp.dot(h2, w3_ref[...], preferred_element_type=jnp.float32)
              + b3_ref[...])

    iota = lax.broadcasted_iota(jnp.int32, (_BT, _E), 1)
    m1 = jnp.max(logits, axis=1, keepdims=True)
    i1 = jnp.min(jnp.where(logits == m1, iota, _E), axis=1, keepdims=True)
    rest = jnp.where(iota == i1, -jnp.inf, logits)
    m2 = jnp.max(rest, axis=1, keepdims=True)
    i2 = jnp.min(jnp.where(rest == m2, iota, _E), axis=1, keepdims=True)
    e2 = jnp.exp(m2 - m1)
    w1v = 1.0 / (1.0 + e2)
    w2v = e2 * w1v
    rw = jnp.where(iota == i1, w1v, jnp.where(iota == i2, w2v, 0.0))
    rw_ref[...] = rw

    usage_ref[...] += jnp.sum(rw, axis=0, keepdims=True)
    conf_ref[...] += jnp.sum(w1v, axis=0, keepdims=True)

    @pl.when(step == _GRID - 1)
    def _finalize():
        usage = usage_ref[...]
        probs = usage * (1.0 / _NTOK)
        util_ref[...] = probs
        d = usage - (_NTOK / _E)
        lbl_ref[...] = jnp.sum(d * d, axis=1, keepdims=True) * (_LBW / _E)
        ent_ref[...] = -jnp.sum(probs * jnp.log(probs + 1e-8), axis=1,
                                keepdims=True)
        conf_ref[...] = conf_ref[...] * (1.0 / _NTOK)


def kernel(hidden_states, W1, b1, W2, b2, W3, b3):
    x = hidden_states.reshape(_NTOK, _H)
    b1r = b1.reshape(1, _RH)
    b2r = b2.reshape(1, _RH2)
    b3r = b3.reshape(1, _E)
    out_shape = (
        jax.ShapeDtypeStruct((_NTOK, _E), jnp.float32),
        jax.ShapeDtypeStruct((1, _E), jnp.float32),
        jax.ShapeDtypeStruct((1, 1), jnp.float32),
        jax.ShapeDtypeStruct((1, 1), jnp.float32),
        jax.ShapeDtypeStruct((1, 1), jnp.float32),
        jax.ShapeDtypeStruct((1, _E), jnp.float32),
    )
    in_specs = [
        pl.BlockSpec((_BT, _H), lambda i: (i, 0)),
        pl.BlockSpec((_H, _RH), lambda i: (0, 0)),
        pl.BlockSpec((1, _RH), lambda i: (0, 0)),
        pl.BlockSpec((_RH, _RH2), lambda i: (0, 0)),
        pl.BlockSpec((1, _RH2), lambda i: (0, 0)),
        pl.BlockSpec((_RH2, _E), lambda i: (0, 0)),
        pl.BlockSpec((1, _E), lambda i: (0, 0)),
    ]
    out_specs = (
        pl.BlockSpec((_BT, _E), lambda i: (i, 0)),
        pl.BlockSpec((1, _E), lambda i: (0, 0)),
        pl.BlockSpec((1, 1), lambda i: (0, 0)),
        pl.BlockSpec((1, 1), lambda i: (0, 0)),
        pl.BlockSpec((1, 1), lambda i: (0, 0)),
        pl.BlockSpec((1, _E), lambda i: (0, 0)),
    )
    rw, usage, conf, lbl, ent, util = pl.pallas_call(
        _router_kernel,
        grid=(_GRID,),
        in_specs=in_specs,
        out_specs=out_specs,
        out_shape=out_shape,
        compiler_params=pltpu.CompilerParams(
            dimension_semantics=("arbitrary",)),
    )(x, W1, b1r, W2, b2r, W3, b3r)
    return (rw.reshape(_B, _S, _E), lbl.reshape(()), ent.reshape(()),
            util.reshape(_E), conf.reshape(()), usage.reshape(_E))


# speedup vs baseline: 1.0813x; 1.0097x over previous
import jax
import jax.numpy as jnp
from jax import lax
from jax.experimental import pallas as pl
from jax.experimental.pallas import tpu as pltpu

_B, _S, _H = 4, 4096, 4096
_E = 64
_RH = 512
_RH2 = 256
_LBW = 0.01
_NTOK = _B * _S
_BT = 1024
_GRID = _NTOK // _BT


def _router_kernel(x_ref, w1_ref, b1_ref, w2_ref, b2_ref, w3_ref, b3_ref,
                   rw_ref, usage_ref, conf_ref, lbl_ref, ent_ref, util_ref,
                   logits_ref):
    step = pl.program_id(0)

    @pl.when(step == 0)
    def _init():
        usage_ref[...] = jnp.zeros_like(usage_ref)
        conf_ref[...] = jnp.zeros_like(conf_ref)
        logits_ref[...] = jnp.zeros_like(logits_ref)

    valid = jnp.where(step > 0, 1.0, 0.0)
    logits = logits_ref[...]
    iota = lax.broadcasted_iota(jnp.int32, (_BT, _E), 1)
    m1 = jnp.max(logits, axis=1, keepdims=True)
    i1 = jnp.min(jnp.where(logits == m1, iota, _E), axis=1, keepdims=True)
    rest = jnp.where(iota == i1, -jnp.inf, logits)
    m2 = jnp.max(rest, axis=1, keepdims=True)
    i2 = jnp.min(jnp.where(rest == m2, iota, _E), axis=1, keepdims=True)
    e2 = jnp.exp(m2 - m1)
    w1v = 1.0 / (1.0 + e2)
    w2v = e2 * w1v
    rw = jnp.where(iota == i1, w1v, jnp.where(iota == i2, w2v, 0.0))
    rw_ref[...] = rw
    usage_ref[...] += valid * jnp.sum(rw, axis=0, keepdims=True)
    conf_ref[...] += valid * jnp.sum(w1v, axis=0, keepdims=True)

    x = x_ref[...]
    h1 = jnp.maximum(
        jnp.dot(x, w1_ref[...], preferred_element_type=jnp.float32)
        + b1_ref[...], 0.0)
    h2 = jnp.maximum(
        jnp.dot(h1, w2_ref[...], preferred_element_type=jnp.float32)
        + b2_ref[...], 0.0)
    logits_ref[...] = (
        jnp.dot(h2, w3_ref[...], preferred_element_type=jnp.float32)
        + b3_ref[...])

    @pl.when(step == _GRID)
    def _finalize():
        usage = usage_ref[...]
        probs = usage * (1.0 / _NTOK)
        util_ref[...] = probs
        d = usage - (_NTOK / _E)
        lbl_ref[...] = jnp.sum(d * d, axis=1, keepdims=True) * (_LBW / _E)
        ent_ref[...] = -jnp.sum(probs * jnp.log(probs + 1e-8), axis=1,
                                keepdims=True)
        conf_ref[...] = conf_ref[...] * (1.0 / _NTOK)


def kernel(hidden_states, W1, b1, W2, b2, W3, b3):
    x = hidden_states.reshape(_NTOK, _H)
    b1r = b1.reshape(1, _RH)
    b2r = b2.reshape(1, _RH2)
    b3r = b3.reshape(1, _E)
    out_shape = (
        jax.ShapeDtypeStruct((_NTOK, _E), jnp.float32),
        jax.ShapeDtypeStruct((1, _E), jnp.float32),
        jax.ShapeDtypeStruct((1, 1), jnp.float32),
        jax.ShapeDtypeStruct((1, 1), jnp.float32),
        jax.ShapeDtypeStruct((1, 1), jnp.float32),
        jax.ShapeDtypeStruct((1, _E), jnp.float32),
    )
    last = _GRID - 1
    in_specs = [
        pl.BlockSpec((_BT, _H), lambda i: (jnp.minimum(i, last), 0)),
        pl.BlockSpec((_H, _RH), lambda i: (0, 0)),
        pl.BlockSpec((1, _RH), lambda i: (0, 0)),
        pl.BlockSpec((_RH, _RH2), lambda i: (0, 0)),
        pl.BlockSpec((1, _RH2), lambda i: (0, 0)),
        pl.BlockSpec((_RH2, _E), lambda i: (0, 0)),
        pl.BlockSpec((1, _E), lambda i: (0, 0)),
    ]
    out_specs = (
        pl.BlockSpec((_BT, _E), lambda i: (jnp.maximum(i - 1, 0), 0)),
        pl.BlockSpec((1, _E), lambda i: (0, 0)),
        pl.BlockSpec((1, 1), lambda i: (0, 0)),
        pl.BlockSpec((1, 1), lambda i: (0, 0)),
        pl.BlockSpec((1, 1), lambda i: (0, 0)),
        pl.BlockSpec((1, _E), lambda i: (0, 0)),
    )
    rw, usage, conf, lbl, ent, util = pl.pallas_call(
        _router_kernel,
        grid=(_GRID + 1,),
        in_specs=in_specs,
        out_specs=out_specs,
        out_shape=out_shape,
        scratch_shapes=[pltpu.VMEM((_BT, _E), jnp.float32)],
        compiler_params=pltpu.CompilerParams(
            dimension_semantics=("arbitrary",)),
    )(x, W1, b1r, W2, b2r, W3, b3r)
    return (rw.reshape(_B, _S, _E), lbl.reshape(()), ent.reshape(()),
            util.reshape(_E), conf.reshape(()), usage.reshape(_E))
